# baseline (speedup 1.0000x reference)
import functools

import jax
import jax.numpy as jnp
from jax import lax
from jax.experimental import pallas as pl
from jax.experimental.pallas import tpu as pltpu
from jax.experimental.pallas import tpu_sc as plsc

VOCAB = 1000
BATCH = 1024
SEQ = 50
NUM_CORES = 2
NUM_SUBCORES = 16
NW = NUM_CORES * NUM_SUBCORES
NVT = 125
BIG = 29
CHUNK_W = 4 * 8 * 8 * 128
ROW_W = NVT * 8 * 8 * 128

_MESH = plsc.VectorSubcoreMesh(core_axis_name="c", subcore_axis_name="s")


@functools.partial(
    pl.kernel,
    out_type=jax.ShapeDtypeStruct((SEQ, ROW_W), jnp.float32),
    mesh=_MESH,
    compiler_params=pltpu.CompilerParams(
        use_tc_tiling_on_sc=False, needs_layout_passes=False
    ),
    scratch_types=[
        pltpu.VMEM((VOCAB, 33), jnp.float32),
        pltpu.VMEM((CHUNK_W,), jnp.float32),
        pltpu.VMEM((CHUNK_W,), jnp.float32),
        pltpu.VMEM((BATCH,), jnp.int32),
        pltpu.VMEM((BATCH,), jnp.int32),
        pltpu.SemaphoreType.DMA,
        pltpu.SemaphoreType.DMA,
        pltpu.SemaphoreType.DMA,
        pltpu.SemaphoreType.DMA,
    ],
)
def _gather_kernel(idxT_hbm, table_hbm, out_hbm, loc, chunk0, chunk1,
                   idx0, idx1, si0, si1, so0, so1):
    wid = lax.axis_index("s") * NUM_CORES + lax.axis_index("c")
    big = wid < BIG
    vt0 = jnp.where(big, 4 * wid, 3 * wid + BIG)
    col0 = vt0 * 8
    obase = vt0 * (8 * 8 * 128)

    chunks = (chunk0, chunk1)
    idxs = (idx0, idx1)
    isems = (si0, si1)
    osems = (so0, so1)

    pltpu.async_copy(idxT_hbm.at[0], idx0, si0)
    pltpu.async_copy(idxT_hbm.at[1], idx1, si1)

    @pl.when(big)
    def _():
        pltpu.sync_copy(table_hbm.at[:, pl.ds(col0, 32)], loc.at[:, pl.ds(0, 32)])

    @pl.when(jnp.logical_not(big))
    def _():
        pltpu.sync_copy(table_hbm.at[:, pl.ds(col0, 24)], loc.at[:, pl.ds(0, 24)])

    def start_out(ch, sem, s):
        @pl.when(big)
        def _():
            pltpu.async_copy(ch, out_hbm.at[s, pl.ds(obase, CHUNK_W)], sem)

        @pl.when(jnp.logical_not(big))
        def _():
            pltpu.async_copy(
                ch.at[pl.ds(0, 3 * 8192)],
                out_hbm.at[s, pl.ds(obase, 3 * 8192)],
                sem,
            )

    def wait_out(ch, sem, s):
        @pl.when(big)
        def _():
            pltpu.make_async_copy(
                ch, out_hbm.at[s, pl.ds(obase, CHUNK_W)], sem
            ).wait()

        @pl.when(jnp.logical_not(big))
        def _():
            pltpu.make_async_copy(
                ch.at[pl.ds(0, 3 * 8192)],
                out_hbm.at[s, pl.ds(obase, 3 * 8192)],
                sem,
            ).wait()

    @pl.loop(0, SEQ // 2)
    def souter(so):
        for sb in range(2):
            s = so * 2 + sb
            ch, iv = chunks[sb], idxs[sb]
            pltpu.make_async_copy(idxT_hbm.at[s], iv, isems[sb]).wait()

            @pl.when(so > 0)
            def _():
                wait_out(ch, osems[sb], s - 2)

            @plsc.parallel_loop(0, BATCH // 16, unroll=4)
            def b0loop(b0):
                rows = iv[pl.ds(b0 * 16, 16)]
                dyn = (b0 // 8) * 1024 + (b0 % 8) * 16
                for k in range(32):
                    cols = jnp.full((16,), k, jnp.int32)
                    g = plsc.load_gather(loc, [rows, cols])
                    koff = (k // 8) * 8192 + (k % 8) * 128
                    ch[pl.ds(dyn + koff, 16)] = g

            @pl.when(s + 2 < SEQ)
            def _():
                pltpu.async_copy(idxT_hbm.at[s + 2], iv, isems[sb])

            start_out(ch, osems[sb], s)

    wait_out(chunk0, so0, SEQ - 2)
    wait_out(chunk1, so1, SEQ - 1)


def kernel(idx, table):
    idx_t = idx.astype(jnp.int32).T
    out = _gather_kernel(idx_t, table)
    return (
        out.reshape(SEQ, NVT, 8, 8, 128)
        .transpose(2, 4, 0, 1, 3)
        .reshape(BATCH, SEQ, VOCAB)
    )


# speedup vs baseline: 1.7292x; 1.7292x over previous
import functools

import jax
import jax.numpy as jnp
from jax import lax
from jax.experimental import pallas as pl
from jax.experimental.pallas import tpu as pltpu
from jax.experimental.pallas import tpu_sc as plsc

VOCAB = 1000
BATCH = 1024
SEQ = 50
NUM_CORES = 2
NUM_SUBCORES = 16
NW = NUM_CORES * NUM_SUBCORES
NVT = 125
BIG = 29
CHUNK_W = 4 * 8 * 8 * 128
ROW_W = NVT * 8 * 8 * 128

_MESH = plsc.VectorSubcoreMesh(core_axis_name="c", subcore_axis_name="s")


@functools.partial(
    pl.kernel,
    out_type=jax.ShapeDtypeStruct((SEQ, ROW_W), jnp.float32),
    mesh=_MESH,
    compiler_params=pltpu.CompilerParams(
        use_tc_tiling_on_sc=False, needs_layout_passes=False
    ),
    scratch_types=[
        pltpu.VMEM((VOCAB, 33), jnp.float32),
        pltpu.VMEM((CHUNK_W,), jnp.float32),
        pltpu.VMEM((CHUNK_W,), jnp.float32),
        pltpu.VMEM((BATCH,), jnp.int32),
        pltpu.VMEM((BATCH,), jnp.int32),
        pltpu.SemaphoreType.DMA,
        pltpu.SemaphoreType.DMA,
        pltpu.SemaphoreType.DMA,
        pltpu.SemaphoreType.DMA,
    ],
)
def _gather_kernel(idxT_hbm, table_hbm, out_hbm, loc, chunk0, chunk1,
                   idx0, idx1, si0, si1, so0, so1):
    wid = lax.axis_index("s") * NUM_CORES + lax.axis_index("c")
    big = wid < BIG
    vt0 = jnp.where(big, 4 * wid, 3 * wid + BIG)
    col0 = vt0 * 8
    obase = vt0 * (8 * 8 * 128)

    chunks = (chunk0, chunk1)
    idxs = (idx0, idx1)
    isems = (si0, si1)
    osems = (so0, so1)

    pltpu.async_copy(idxT_hbm.at[0], idx0, si0)
    pltpu.async_copy(idxT_hbm.at[1], idx1, si1)

    @pl.when(big)
    def _():
        pltpu.sync_copy(table_hbm.at[:, pl.ds(col0, 32)], loc.at[:, pl.ds(0, 32)])

    @pl.when(jnp.logical_not(big))
    def _():
        pltpu.sync_copy(table_hbm.at[:, pl.ds(col0, 24)], loc.at[:, pl.ds(0, 24)])

    def start_out(ch, sem, s):
        @pl.when(big)
        def _():
            pltpu.async_copy(ch, out_hbm.at[s, pl.ds(obase, CHUNK_W)], sem)

        @pl.when(jnp.logical_not(big))
        def _():
            pltpu.async_copy(
                ch.at[pl.ds(0, 3 * 8192)],
                out_hbm.at[s, pl.ds(obase, 3 * 8192)],
                sem,
            )

    def wait_out(ch, sem, s):
        @pl.when(big)
        def _():
            pltpu.make_async_copy(
                ch, out_hbm.at[s, pl.ds(obase, CHUNK_W)], sem
            ).wait()

        @pl.when(jnp.logical_not(big))
        def _():
            pltpu.make_async_copy(
                ch.at[pl.ds(0, 3 * 8192)],
                out_hbm.at[s, pl.ds(obase, 3 * 8192)],
                sem,
            ).wait()

    @pl.loop(0, SEQ // 2)
    def souter(so):
        for sb in range(2):
            s = so * 2 + sb
            ch, iv = chunks[sb], idxs[sb]
            pltpu.make_async_copy(idxT_hbm.at[s], iv, isems[sb]).wait()

            @pl.when(so > 0)
            def _():
                wait_out(ch, osems[sb], s - 2)

            @plsc.parallel_loop(0, BATCH // 16, step=2)
            def b0loop(b0):
                for b1 in range(2):
                    bb = b0 + b1
                    rows = iv[pl.ds(bb * 16, 16)]
                    dyn = (bb // 8) * 1024 + (bb % 8) * 16
                    for k in range(32):
                        cols = jnp.full((16,), k, jnp.int32)
                        g = plsc.load_gather(loc, [rows, cols])
                        koff = (k // 8) * 8192 + (k % 8) * 128
                        ch[pl.ds(dyn + koff, 16)] = g

            @pl.when(s + 2 < SEQ)
            def _():
                pltpu.async_copy(idxT_hbm.at[s + 2], iv, isems[sb])

            start_out(ch, osems[sb], s)

    wait_out(chunk0, so0, SEQ - 2)
    wait_out(chunk1, so1, SEQ - 1)


def kernel(idx, table):
    idx_t = idx.astype(jnp.int32).T
    out = _gather_kernel(idx_t, table)
    return (
        out.reshape(SEQ, NVT, 8, 8, 128)
        .transpose(2, 4, 0, 1, 3)
        .reshape(BATCH, SEQ, VOCAB)
    )
